# Initial kernel scaffold; baseline (speedup 1.0000x reference)
import jax
import jax.numpy as jnp
from jax.experimental import pallas as pl


def kernel(xin, W, b):
    raise NotImplementedError("write your pallas kernel here")


# speedup vs baseline: 3.3518x; 3.3518x over previous
import functools

import jax
import jax.numpy as jnp
from jax.experimental import pallas as pl
from jax.experimental.pallas import tpu as pltpu

_STRIDE = 8.0
_ANCHORS = ((10.0, 13.0), (16.0, 30.0), (33.0, 23.0))
_A = 3
_C = 85
_F = 76
_P = _F * _F


def _body(x_ref, w_ref, b_ref, o_ref, *, pt):
    x = x_ref[0]
    w = w_ref[...]
    acc = jax.lax.dot_general(
        x, w, (((0,), (1,)), ((), ())),
        preferred_element_type=jnp.float32,
    )
    acc = acc + b_ref[...]

    rows = jax.lax.broadcasted_iota(jnp.int32, (pt, 128), 0)
    col = jax.lax.broadcasted_iota(jnp.int32, (pt, 128), 1)
    xs = (rows % _F).astype(jnp.float32)
    ys = (rows // _F).astype(jnp.float32)

    for a in range(_A):
        t = acc[:, a * 128:(a + 1) * 128]
        sig = jax.nn.sigmoid(t)
        ex = jnp.exp(t)
        aw, ah = _ANCHORS[a]
        val = jnp.where(col == 0, (sig + xs) * _STRIDE,
              jnp.where(col == 1, (sig + ys) * _STRIDE,
              jnp.where(col == 2, ex * aw,
              jnp.where(col == 3, ex * ah, sig))))
        o_ref[0, a] = val[:, :_C]


def kernel(xin, W, b):
    N = xin.shape[0]
    x3 = xin.reshape(N, 128, _P)
    w3 = W.reshape(_A, _C, 128)
    wp = jnp.pad(w3, ((0, 0), (0, 128 - _C), (0, 0))).reshape(_A * 128, 128)
    bp = jnp.pad(b.reshape(_A, _C), ((0, 0), (0, 128 - _C))).reshape(1, _A * 128)

    pt = _P
    grid = (N,)
    out = pl.pallas_call(
        functools.partial(_body, pt=pt),
        grid=grid,
        in_specs=[
            pl.BlockSpec((1, 128, pt), lambda n: (n, 0, 0)),
            pl.BlockSpec((_A * 128, 128), lambda n: (0, 0)),
            pl.BlockSpec((1, _A * 128), lambda n: (0, 0)),
        ],
        out_specs=pl.BlockSpec((1, _A, pt, _C), lambda n: (n, 0, 0, 0)),
        out_shape=jax.ShapeDtypeStruct((N, _A, _P, _C), jnp.float32),
        compiler_params=pltpu.CompilerParams(
            dimension_semantics=("parallel",),
        ),
    )(x3, wp, bp)
    return out.reshape(N, _A * _P, _C)


# speedup vs baseline: 4.0200x; 1.1994x over previous
import jax
import jax.numpy as jnp
from jax.experimental import pallas as pl
from jax.experimental.pallas import tpu as pltpu

_STRIDE = 8.0
_ANCHORS = ((10.0, 13.0), (16.0, 30.0), (33.0, 23.0))
_A = 3
_C = 85
_F = 76
_P = _F * _F
_PP = 5888
_T = 256
_NSTEP = (_P + _T - 1) // _T
_N = 16


def _body(x_ref, w_ref, b_ref, o_ref):
    w = w_ref[...]
    b = b_ref[...]

    base = pl.program_id(0) * _T
    lanes = jax.lax.broadcasted_iota(jnp.int32, (_C + 3, _T), 1) + base
    sub = jax.lax.broadcasted_iota(jnp.int32, (_C + 3, _T), 0)
    xs8 = (lanes % _F).astype(jnp.float32) * _STRIDE
    ys8 = (lanes // _F).astype(jnp.float32) * _STRIDE
    shift8 = jnp.where(sub == 0, xs8, jnp.where(sub == 1, ys8, 0.0))
    m_wh = (sub == 2) | (sub == 3)
    sign = jnp.where(m_wh, 1.0, -1.0)
    scale = jnp.where(sub < 2, _STRIDE, 1.0)
    anchs = [jnp.where(sub == 2, aw, ah) for aw, ah in _ANCHORS]

    for n in range(_N):
        xb = x_ref[:, n, :]
        acc = jax.lax.dot_general(
            w, xb, (((1,), (1,)), ((), ())),
            preferred_element_type=jnp.float32,
        ) + b
        for a in range(_A):
            t = acc[a * 128:a * 128 + _C + 3, :]
            e = jnp.exp(t * sign)
            sig = 1.0 / (1.0 + e)
            val = jnp.where(m_wh, e * anchs[a], sig * scale + shift8)
            o_ref[:, n, a, :] = val[:_C, :]


def kernel(xin, W, b):
    N = xin.shape[0]
    xt = jnp.transpose(xin, (2, 3, 0, 1)).reshape(_P, N, 128)
    w3 = W.reshape(_A, _C, 128)
    wp = jnp.pad(w3, ((0, 0), (0, 128 - _C), (0, 0))).reshape(_A * 128, 128)
    bp = jnp.pad(b.reshape(_A, _C), ((0, 0), (0, 128 - _C))).reshape(_A * 128, 1)

    out = pl.pallas_call(
        _body,
        grid=(_NSTEP,),
        in_specs=[
            pl.BlockSpec((_T, N, 128), lambda k: (k, 0, 0)),
            pl.BlockSpec((_A * 128, 128), lambda k: (0, 0)),
            pl.BlockSpec((_A * 128, 1), lambda k: (0, 0)),
        ],
        out_specs=pl.BlockSpec((_C, N, _A, _T), lambda k: (0, 0, 0, k)),
        out_shape=jax.ShapeDtypeStruct((_C, N, _A, _PP), jnp.float32),
        compiler_params=pltpu.CompilerParams(
            dimension_semantics=("parallel",),
        ),
    )(xt, wp, bp)
    return out[:, :, :, :_P].reshape(_C, N, _A * _P).transpose(1, 2, 0)


# speedup vs baseline: 4.8301x; 1.2015x over previous
import jax
import jax.numpy as jnp
from jax.experimental import pallas as pl
from jax.experimental.pallas import tpu as pltpu

_STRIDE = 8.0
_ANCHORS = ((10.0, 13.0), (16.0, 30.0), (33.0, 23.0))
_A = 3
_C = 85
_F = 76
_P = _F * _F
_T = 304
_KS = _P // _T
_N = 16


def _body(x_ref, w_ref, b_ref, o_ref):
    w = w_ref[...]
    b = b_ref[...]

    base = pl.program_id(0) * _T
    rows = jax.lax.broadcasted_iota(jnp.int32, (_T, 128), 0) + base
    col = jax.lax.broadcasted_iota(jnp.int32, (_T, 128), 1)
    xs8 = (rows % _F).astype(jnp.float32) * _STRIDE
    ys8 = (rows // _F).astype(jnp.float32) * _STRIDE
    shift8 = jnp.where(col == 0, xs8, jnp.where(col == 1, ys8, 0.0))
    m_wh = (col == 2) | (col == 3)
    sign = jnp.where(m_wh, 1.0, -1.0)
    scale = jnp.where(col < 2, _STRIDE, 1.0)
    anchs = [jnp.where(col == 2, aw, ah) for aw, ah in _ANCHORS]

    for n in range(_N):
        xb = x_ref[:, n, :]
        acc = jax.lax.dot_general(
            xb, w, (((1,), (1,)), ((), ())),
            preferred_element_type=jnp.float32,
        ) + b
        for a in range(_A):
            t = acc[:, a * 128:(a + 1) * 128]
            e = jnp.exp(t * sign)
            sig = 1.0 / (1.0 + e)
            val = jnp.where(m_wh, e * anchs[a], sig * scale + shift8)
            o_ref[n, a, :, :] = val[:, :_C]


def kernel(xin, W, b):
    N = xin.shape[0]
    xt = jnp.transpose(xin, (2, 3, 0, 1)).reshape(_P, N, 128)
    w3 = W.reshape(_A, _C, 128)
    wp = jnp.pad(w3, ((0, 0), (0, 128 - _C), (0, 0))).reshape(_A * 128, 128)
    bp = jnp.pad(b.reshape(_A, _C), ((0, 0), (0, 128 - _C))).reshape(1, _A * 128)

    out = pl.pallas_call(
        _body,
        grid=(_KS,),
        in_specs=[
            pl.BlockSpec((_T, N, 128), lambda k: (k, 0, 0)),
            pl.BlockSpec((_A * 128, 128), lambda k: (0, 0)),
            pl.BlockSpec((1, _A * 128), lambda k: (0, 0)),
        ],
        out_specs=pl.BlockSpec((N, _A, _T, _C), lambda k: (0, 0, k, 0)),
        out_shape=jax.ShapeDtypeStruct((N, _A, _P, _C), jnp.float32),
        compiler_params=pltpu.CompilerParams(
            dimension_semantics=("parallel",),
        ),
    )(xt, wp, bp)
    return out.reshape(N, _A * _P, _C)
